# Initial kernel scaffold; baseline (speedup 1.0000x reference)
import functools

import jax
import jax.numpy as jnp
from jax import lax
from jax.experimental import pallas as pl
from jax.experimental.pallas import tpu as pltpu
from jax.experimental.pallas import tpu_sc as plsc

N = 10000
E = 320000
NPAD = 10240
NTILE = 32
EPT = E // NTILE
CH = 80
R = 1000
ROWS_PER_TILE = NPAD // 16


def _f32(*shape):
    return jax.ShapeDtypeStruct(shape, jnp.float32)


def _tc_first_body(x_ref, wsrc_ref, wscp_ref, linw_ref, xs_ref, sc_ref, hlin_ref):
    h = x_ref[...]
    xs_ref[...] = jnp.dot(h, wsrc_ref[...], preferred_element_type=jnp.float32)
    sc_ref[...] = jnp.dot(h, wscp_ref[...], preferred_element_type=jnp.float32)
    hlin_ref[...] = jnp.dot(h, linw_ref[...], preferred_element_type=jnp.float32)


def _finalize(msg_ref, den_ref, hlinp_ref, b_ref):
    m = msg_ref[0] + msg_ref[1]
    dn = jnp.sum(den_ref[...], axis=0)
    return m / (dn[:, None] + 1e-30) + b_ref[...] + hlinp_ref[...]


def _tc_mid_body(msg_ref, den_ref, hlinp_ref, b_ref, wsrc_ref, wscp_ref,
                 linw_ref, xs_ref, sc_ref, hlin_ref):
    h = jnp.maximum(_finalize(msg_ref, den_ref, hlinp_ref, b_ref), 0.0)
    xs_ref[...] = jnp.dot(h, wsrc_ref[...], preferred_element_type=jnp.float32)
    sc_ref[...] = jnp.dot(h, wscp_ref[...], preferred_element_type=jnp.float32)
    hlin_ref[...] = jnp.dot(h, linw_ref[...], preferred_element_type=jnp.float32)


def _tc_final_body(msg_ref, den_ref, hlinp_ref, b_ref, out_ref):
    out_ref[...] = jax.nn.sigmoid(_finalize(msg_ref, den_ref, hlinp_ref, b_ref))


def _tc_first(x, wsrc, wscp, linw, do):
    di = x.shape[1]
    return pl.pallas_call(
        _tc_first_body,
        grid=(N // R,),
        in_specs=[
            pl.BlockSpec((R, di), lambda i: (i, 0)),
            pl.BlockSpec((di, do), lambda i: (0, 0)),
            pl.BlockSpec((di, 128), lambda i: (0, 0)),
            pl.BlockSpec((di, do), lambda i: (0, 0)),
        ],
        out_specs=[
            pl.BlockSpec((R, do), lambda i: (i, 0)),
            pl.BlockSpec((R, 128), lambda i: (i, 0)),
            pl.BlockSpec((R, do), lambda i: (i, 0)),
        ],
        out_shape=[_f32(N, do), _f32(N, 128), _f32(N, do)],
    )(x, wsrc, wscp, linw)


def _tc_mid(msg, den, hlinp, b, wsrc, wscp, linw, dp, do):
    di = dp
    return pl.pallas_call(
        _tc_mid_body,
        grid=(N // R,),
        in_specs=[
            pl.BlockSpec((2, R, dp), lambda i: (0, i, 0)),
            pl.BlockSpec((NTILE, R), lambda i: (0, i)),
            pl.BlockSpec((R, dp), lambda i: (i, 0)),
            pl.BlockSpec((1, dp), lambda i: (0, 0)),
            pl.BlockSpec((di, do), lambda i: (0, 0)),
            pl.BlockSpec((di, 128), lambda i: (0, 0)),
            pl.BlockSpec((di, do), lambda i: (0, 0)),
        ],
        out_specs=[
            pl.BlockSpec((R, do), lambda i: (i, 0)),
            pl.BlockSpec((R, 128), lambda i: (i, 0)),
            pl.BlockSpec((R, do), lambda i: (i, 0)),
        ],
        out_shape=[_f32(N, do), _f32(N, 128), _f32(N, do)],
    )(msg, den, hlinp, b, wsrc, wscp, linw)


def _tc_final(msg, den, hlinp, b, dp):
    return pl.pallas_call(
        _tc_final_body,
        grid=(N // R,),
        in_specs=[
            pl.BlockSpec((2, R, dp), lambda i: (0, i, 0)),
            pl.BlockSpec((NTILE, R), lambda i: (0, i)),
            pl.BlockSpec((R, dp), lambda i: (i, 0)),
            pl.BlockSpec((1, dp), lambda i: (0, 0)),
        ],
        out_specs=pl.BlockSpec((R, dp), lambda i: (i, 0)),
        out_shape=_f32(N, dp),
    )(msg, den, hlinp, b)


def _escore_body(ea_ref, bd_ref, out_ref):
    out_ref[...] = jnp.dot(ea_ref[...], bd_ref[...],
                           preferred_element_type=jnp.float32)


def _escore(ea2, bd):
    rows = ea2.shape[0]
    return pl.pallas_call(
        _escore_body,
        grid=(rows // R,),
        in_specs=[
            pl.BlockSpec((R, 128), lambda i: (i, 0)),
            pl.BlockSpec((128, 32), lambda i: (0, 0)),
        ],
        out_specs=pl.BlockSpec((R, 32), lambda i: (i, 0)),
        out_shape=_f32(rows, 32),
    )(ea2, bd)


def _sc_body(do, src_h, dst_h, el_h, ssrc_h, sdst_h, xs_h, msg_o, den_o,
             ssrc_v, sdst_v, den_v, src_c, dst_c, el_c, ex_c, rows_v,
             acc_s, sem):
    nc = do // 16
    cid = lax.axis_index("c")
    sid = lax.axis_index("s")
    w = sid * 2 + cid
    ebase = w * EPT

    pltpu.sync_copy(ssrc_h, ssrc_v)
    pltpu.sync_copy(sdst_h, sdst_v)

    zero16 = jnp.zeros((16,), jnp.float32)

    def _zden(k, carry):
        den_v[pl.ds(k * 16, 16)] = zero16
        return carry
    lax.fori_loop(0, NPAD // 16, _zden, 0)

    def _zrow(k, carry):
        rr = k // nc
        cc = (k - rr * nc) * 16
        rows_v[rr, pl.ds(cc, 16)] = zero16
        return carry
    lax.fori_loop(0, CH * nc, _zrow, 0)
    for k in range(ROWS_PER_TILE // CH):
        pltpu.sync_copy(rows_v,
                        acc_s.at[pl.ds(sid * ROWS_PER_TILE + k * CH, CH)])
    plsc.subcore_barrier()

    def _chunk(t, carry):
        base = ebase + t * CH
        pltpu.sync_copy(src_h.at[pl.ds(base, CH)], src_c)
        pltpu.sync_copy(dst_h.at[pl.ds(base, CH)], dst_c)
        pltpu.sync_copy(el_h.at[pl.ds(base, CH)], el_c)
        gcp = pltpu.async_copy(xs_h.at[src_c], rows_v, sem)
        for g in range(CH // 16):
            s16 = src_c[pl.ds(g * 16, 16)]
            d16 = dst_c[pl.ds(g * 16, 16)]
            a = (plsc.load_gather(ssrc_v, [s16])
                 + plsc.load_gather(sdst_v, [d16])
                 + el_c[pl.ds(g * 16, 16)])
            a = jnp.where(a >= 0.0, a, 0.2 * a)
            ex = jnp.exp(a)
            ex_c[pl.ds(g * 16, 16)] = ex
            plsc.addupdate_scatter(den_v, [d16], ex)
        gcp.wait()

        def _sgrp(g, c2):
            ex16 = ex_c[pl.ds(g * 16, 16)]

            def _sl(l, c3):
                exj = jnp.take(ex16, jnp.full((16,), l, jnp.int32),
                               mode="promise_in_bounds")
                j = g * 16 + l
                for c in range(nc):
                    rows_v[j, pl.ds(c * 16, 16)] = (
                        rows_v[j, pl.ds(c * 16, 16)] * exj)
                return c3
            return lax.fori_loop(0, 16, _sl, c2)
        lax.fori_loop(0, CH // 16, _sgrp, 0)

        pltpu.sync_copy(rows_v, acc_s.at[dst_c], add=True)
        return carry
    lax.fori_loop(0, EPT // CH, _chunk, 0)

    plsc.subcore_barrier()
    pltpu.sync_copy(acc_s.at[pl.ds(sid * ROWS_PER_TILE, ROWS_PER_TILE)],
                    msg_o.at[cid, pl.ds(sid * ROWS_PER_TILE, ROWS_PER_TILE)])
    pltpu.sync_copy(den_v, den_o.at[w])


def _sc_edge(do, src, dst, el, ssrc, sdst, xs):
    mesh = plsc.VectorSubcoreMesh(core_axis_name="c", subcore_axis_name="s")
    kern = pl.kernel(
        functools.partial(_sc_body, do),
        mesh=mesh,
        out_type=[_f32(2, NPAD, do), _f32(NTILE, NPAD)],
        scratch_types=[
            pltpu.VMEM((N,), jnp.float32),
            pltpu.VMEM((N,), jnp.float32),
            pltpu.VMEM((NPAD,), jnp.float32),
            pltpu.VMEM((CH,), jnp.int32),
            pltpu.VMEM((CH,), jnp.int32),
            pltpu.VMEM((CH,), jnp.float32),
            pltpu.VMEM((CH,), jnp.float32),
            pltpu.VMEM((CH, do), jnp.float32),
            pltpu.VMEM_SHARED((NPAD, do), jnp.float32),
            pltpu.SemaphoreType.DMA,
        ],
    )
    return kern(src, dst, el, ssrc, sdst, xs)


def kernel(x, edge_index, edge_attr,
           conv1_Wsrc, conv1_Wdst, conv1_We, conv1_asrc, conv1_adst, conv1_ae, conv1_b, lin1_W,
           conv2_Wsrc, conv2_Wdst, conv2_We, conv2_asrc, conv2_adst, conv2_ae, conv2_b, lin2_W,
           conv3_Wsrc, conv3_Wdst, conv3_We, conv3_asrc, conv3_adst, conv3_ae, conv3_b, lin3_W,
           conv4_Wsrc, conv4_Wdst, conv4_We, conv4_asrc, conv4_adst, conv4_ae, conv4_b, lin4_W):
    convs = [
        (conv1_Wsrc, conv1_Wdst, conv1_We, conv1_asrc, conv1_adst, conv1_ae, conv1_b, lin1_W),
        (conv2_Wsrc, conv2_Wdst, conv2_We, conv2_asrc, conv2_adst, conv2_ae, conv2_b, lin2_W),
        (conv3_Wsrc, conv3_Wdst, conv3_We, conv3_asrc, conv3_adst, conv3_ae, conv3_b, lin3_W),
        (conv4_Wsrc, conv4_Wdst, conv4_We, conv4_asrc, conv4_adst, conv4_ae, conv4_b, lin4_W),
    ]
    src = edge_index[0]
    dst = edge_index[1]

    wscps = []
    for (Wsrc, Wdst, We, asrc, adst, ae, b, linW) in convs:
        wsa = Wsrc @ asrc
        wdd = Wdst @ adst
        wscp = jnp.zeros((Wsrc.shape[0], 128), jnp.float32)
        wscp = wscp.at[:, 0].set(wsa).at[:, 1].set(wdd)
        wscps.append(wscp)
    blk = jnp.stack([c[2] @ c[5] for c in convs], axis=1)
    bd = jnp.kron(jnp.eye(8, dtype=jnp.float32), blk)

    es4 = _escore(edge_attr.reshape(E // 8, 128), bd).reshape(E, 4)

    dos = [128, 128, 128, 64]
    xs, sc2, hlin = _tc_first(x, convs[0][0], wscps[0], convs[0][7], dos[0])
    msg, den = _sc_edge(dos[0], src, dst, es4[:, 0], sc2[:, 0], sc2[:, 1], xs)
    for i in (1, 2, 3):
        dp, do = dos[i - 1], dos[i]
        b_prev = convs[i - 1][6].reshape(1, dp)
        xs, sc2, hlin = _tc_mid(msg, den, hlin, b_prev,
                                convs[i][0], wscps[i], convs[i][7], dp, do)
        msg, den = _sc_edge(do, src, dst, es4[:, i], sc2[:, 0], sc2[:, 1], xs)
    return _tc_final(msg, den, hlin, convs[3][6].reshape(1, dos[3]), dos[3])


# speedup vs baseline: 17.3685x; 17.3685x over previous
import functools

import jax
import jax.numpy as jnp
from jax import lax
from jax.experimental import pallas as pl
from jax.experimental.pallas import tpu as pltpu
from jax.experimental.pallas import tpu_sc as plsc

N = 10000
E = 320000
NPAD = 10240
NTILE = 32
EPT = E // NTILE
CH = 80
R = 1024
ROWS_PER_TILE = NPAD // 16


def _f32(*shape):
    return jax.ShapeDtypeStruct(shape, jnp.float32)


_GDN = lax.GatherDimensionNumbers(
    offset_dims=(), collapsed_slice_dims=(0,), start_index_map=(0,))


def _bcast_lane(v16, lane):
    idx = jnp.full((16, 1), lane, jnp.int32)
    return lax.gather(v16, idx, _GDN, (1,),
                      mode=lax.GatherScatterMode.PROMISE_IN_BOUNDS)


def _tc_first_body(x_ref, wsrc_ref, wscp_ref, linw_ref, xs_ref, sc_ref, hlin_ref):
    h = x_ref[...]
    xs_ref[...] = jnp.dot(h, wsrc_ref[...], preferred_element_type=jnp.float32)
    sc_ref[...] = jnp.dot(h, wscp_ref[...], preferred_element_type=jnp.float32)
    hlin_ref[...] = jnp.dot(h, linw_ref[...], preferred_element_type=jnp.float32)


def _finalize(msg_ref, den_ref, hlinp_ref, b_ref, dp):
    i = pl.program_id(0)
    m = (msg_ref[0] + msg_ref[1])[:, :dp]
    dn = jnp.sum(den_ref[:, pl.ds(i * R, R)], axis=0)
    return m / (dn[:, None] + 1e-30) + b_ref[...] + hlinp_ref[...]


def _tc_mid_body(dp, msg_ref, den_ref, hlinp_ref, b_ref, wsrc_ref, wscp_ref,
                 linw_ref, xs_ref, sc_ref, hlin_ref):
    h = jnp.maximum(_finalize(msg_ref, den_ref, hlinp_ref, b_ref, dp), 0.0)
    xs_ref[...] = jnp.dot(h, wsrc_ref[...], preferred_element_type=jnp.float32)
    sc_ref[...] = jnp.dot(h, wscp_ref[...], preferred_element_type=jnp.float32)
    hlin_ref[...] = jnp.dot(h, linw_ref[...], preferred_element_type=jnp.float32)


def _tc_final_body(dp, msg_ref, den_ref, hlinp_ref, b_ref, out_ref):
    out_ref[...] = jax.nn.sigmoid(
        _finalize(msg_ref, den_ref, hlinp_ref, b_ref, dp))


def _tc_first(x, wsrc, wscp, linw, do):
    di = x.shape[1]
    return pl.pallas_call(
        _tc_first_body,
        grid=(pl.cdiv(N, R),),
        in_specs=[
            pl.BlockSpec((R, di), lambda i: (i, 0)),
            pl.BlockSpec((di, do), lambda i: (0, 0)),
            pl.BlockSpec((di, 128), lambda i: (0, 0)),
            pl.BlockSpec((di, do), lambda i: (0, 0)),
        ],
        out_specs=[
            pl.BlockSpec((R, do), lambda i: (i, 0)),
            pl.BlockSpec((R, 128), lambda i: (i, 0)),
            pl.BlockSpec((R, do), lambda i: (i, 0)),
        ],
        out_shape=[_f32(N, do), _f32(N, 128), _f32(N, do)],
    )(x, wsrc, wscp, linw)


def _tc_mid(msg, den, hlinp, b, wsrc, wscp, linw, dp, do):
    di = dp
    return pl.pallas_call(
        functools.partial(_tc_mid_body, dp),
        grid=(pl.cdiv(N, R),),
        in_specs=[
            pl.BlockSpec((2, R, 128), lambda i: (0, i, 0)),
            pl.BlockSpec((NTILE, NPAD), lambda i: (0, 0)),
            pl.BlockSpec((R, dp), lambda i: (i, 0)),
            pl.BlockSpec((1, dp), lambda i: (0, 0)),
            pl.BlockSpec((di, 128), lambda i: (0, 0)),
            pl.BlockSpec((di, 128), lambda i: (0, 0)),
            pl.BlockSpec((di, do), lambda i: (0, 0)),
        ],
        out_specs=[
            pl.BlockSpec((R, 128), lambda i: (i, 0)),
            pl.BlockSpec((R, 128), lambda i: (i, 0)),
            pl.BlockSpec((R, do), lambda i: (i, 0)),
        ],
        out_shape=[_f32(N, 128), _f32(N, 128), _f32(N, do)],
    )(msg, den, hlinp, b, wsrc, wscp, linw)


def _tc_final(msg, den, hlinp, b, dp):
    return pl.pallas_call(
        functools.partial(_tc_final_body, dp),
        grid=(pl.cdiv(N, R),),
        in_specs=[
            pl.BlockSpec((2, R, 128), lambda i: (0, i, 0)),
            pl.BlockSpec((NTILE, NPAD), lambda i: (0, 0)),
            pl.BlockSpec((R, dp), lambda i: (i, 0)),
            pl.BlockSpec((1, dp), lambda i: (0, 0)),
        ],
        out_specs=pl.BlockSpec((R, dp), lambda i: (i, 0)),
        out_shape=_f32(N, dp),
    )(msg, den, hlinp, b)


def _escore_body(ea_ref, bd_ref, out_ref):
    out_ref[...] = jnp.dot(ea_ref[...], bd_ref[...],
                           preferred_element_type=jnp.float32)


def _escore(ea2, bd):
    rows = ea2.shape[0]
    return pl.pallas_call(
        _escore_body,
        grid=(pl.cdiv(rows, R),),
        in_specs=[
            pl.BlockSpec((R, 128), lambda i: (i, 0)),
            pl.BlockSpec((128, 32), lambda i: (0, 0)),
        ],
        out_specs=pl.BlockSpec((R, 32), lambda i: (i, 0)),
        out_shape=_f32(rows, 32),
    )(ea2, bd)


def _sc_body(do, src_h, dst_h, el_h, ssrc_h, sdst_h, xs_h, msg_o, den_o,
             ssrc_v, sdst_v, den_v, src_c, dst_c, el_c, ex_c, rows_v,
             acc_s, sem):
    nc = do // 16
    cid = lax.axis_index("c")
    sid = lax.axis_index("s")
    w = sid * 2 + cid
    ebase = w * EPT

    pltpu.sync_copy(ssrc_h, ssrc_v)
    pltpu.sync_copy(sdst_h, sdst_v)

    zero16 = jnp.zeros((16,), jnp.float32)

    def _zden(k, carry):
        den_v[pl.ds(k * 16, 16)] = zero16
        return carry
    lax.fori_loop(0, NPAD // 16, _zden, 0)

    def _zrow(k, carry):
        rr = k // nc
        cc = (k - rr * nc) * 16
        rows_v[rr, pl.ds(cc, 16)] = zero16
        return carry
    lax.fori_loop(0, CH * nc, _zrow, 0)
    for k in range(ROWS_PER_TILE // CH):
        pltpu.sync_copy(rows_v,
                        acc_s.at[pl.ds(sid * ROWS_PER_TILE + k * CH, CH)])
    plsc.subcore_barrier()

    def _chunk(t, carry):
        base = ebase + t * CH
        pltpu.sync_copy(src_h.at[pl.ds(base, CH)], src_c)
        pltpu.sync_copy(dst_h.at[pl.ds(base, CH)], dst_c)
        pltpu.sync_copy(el_h.at[pl.ds(base, CH)], el_c)
        gcp = pltpu.async_copy(xs_h.at[src_c], rows_v, sem)
        for g in range(CH // 16):
            s16 = src_c[pl.ds(g * 16, 16)]
            d16 = dst_c[pl.ds(g * 16, 16)]
            a = (plsc.load_gather(ssrc_v, [s16])
                 + plsc.load_gather(sdst_v, [d16])
                 + el_c[pl.ds(g * 16, 16)])
            a = jnp.where(a >= 0.0, a, 0.2 * a)
            ex = jnp.exp(a)
            ex_c[pl.ds(g * 16, 16)] = ex
            plsc.addupdate_scatter(den_v, [d16], ex)
        gcp.wait()

        def _sgrp(g, c2):
            ex16 = ex_c[pl.ds(g * 16, 16)]

            def _sl(l, c3):
                exj = _bcast_lane(ex16, l)
                j = g * 16 + l
                for c in range(nc):
                    rows_v[j, pl.ds(c * 16, 16)] = (
                        rows_v[j, pl.ds(c * 16, 16)] * exj)
                return c3
            return lax.fori_loop(0, 16, _sl, c2)
        lax.fori_loop(0, CH // 16, _sgrp, 0)

        pltpu.sync_copy(rows_v, acc_s.at[dst_c], add=True)
        return carry
    lax.fori_loop(0, EPT // CH, _chunk, 0)

    plsc.subcore_barrier()
    pltpu.sync_copy(acc_s.at[pl.ds(sid * ROWS_PER_TILE, ROWS_PER_TILE)],
                    msg_o.at[cid, pl.ds(sid * ROWS_PER_TILE, ROWS_PER_TILE)])
    pltpu.sync_copy(den_v, den_o.at[w])


def _sc_edge(do, src, dst, el, ssrc, sdst, xs):
    mesh = plsc.VectorSubcoreMesh(core_axis_name="c", subcore_axis_name="s")
    kern = pl.kernel(
        functools.partial(_sc_body, do),
        mesh=mesh,
        compiler_params=pltpu.CompilerParams(needs_layout_passes=False),
        out_type=[_f32(2, NPAD, do), _f32(NTILE, NPAD)],
        scratch_types=[
            pltpu.VMEM((N,), jnp.float32),
            pltpu.VMEM((N,), jnp.float32),
            pltpu.VMEM((NPAD,), jnp.float32),
            pltpu.VMEM((CH,), jnp.int32),
            pltpu.VMEM((CH,), jnp.int32),
            pltpu.VMEM((CH,), jnp.float32),
            pltpu.VMEM((CH,), jnp.float32),
            pltpu.VMEM((CH, do), jnp.float32),
            pltpu.VMEM_SHARED((NPAD, do), jnp.float32),
            pltpu.SemaphoreType.DMA,
        ],
    )
    return kern(src, dst, el, ssrc, sdst, xs)


def kernel(x, edge_index, edge_attr,
           conv1_Wsrc, conv1_Wdst, conv1_We, conv1_asrc, conv1_adst, conv1_ae, conv1_b, lin1_W,
           conv2_Wsrc, conv2_Wdst, conv2_We, conv2_asrc, conv2_adst, conv2_ae, conv2_b, lin2_W,
           conv3_Wsrc, conv3_Wdst, conv3_We, conv3_asrc, conv3_adst, conv3_ae, conv3_b, lin3_W,
           conv4_Wsrc, conv4_Wdst, conv4_We, conv4_asrc, conv4_adst, conv4_ae, conv4_b, lin4_W):
    convs = [
        (conv1_Wsrc, conv1_Wdst, conv1_We, conv1_asrc, conv1_adst, conv1_ae, conv1_b, lin1_W),
        (conv2_Wsrc, conv2_Wdst, conv2_We, conv2_asrc, conv2_adst, conv2_ae, conv2_b, lin2_W),
        (conv3_Wsrc, conv3_Wdst, conv3_We, conv3_asrc, conv3_adst, conv3_ae, conv3_b, lin3_W),
        (conv4_Wsrc, conv4_Wdst, conv4_We, conv4_asrc, conv4_adst, conv4_ae, conv4_b, lin4_W),
    ]
    src = edge_index[0]
    dst = edge_index[1]

    wscps = []
    for (Wsrc, Wdst, We, asrc, adst, ae, b, linW) in convs:
        wsa = Wsrc @ asrc
        wdd = Wdst @ adst
        wscp = jnp.zeros((Wsrc.shape[0], 128), jnp.float32)
        wscp = wscp.at[:, 0].set(wsa).at[:, 1].set(wdd)
        wscps.append(wscp)
    blk = jnp.stack([c[2] @ c[5] for c in convs], axis=1)
    bd = jnp.kron(jnp.eye(8, dtype=jnp.float32), blk)

    es4 = _escore(edge_attr.reshape(E // 8, 128), bd).reshape(E, 4)

    dos = [128, 128, 128, 64]
    xs, sc2, hlin = _tc_first(x, convs[0][0], wscps[0], convs[0][7], 128)
    msg, den = _sc_edge(128, src, dst, es4[:, 0], sc2[:, 0], sc2[:, 1], xs)
    for i in (1, 2, 3):
        dp, do = dos[i - 1], dos[i]
        b_prev = convs[i - 1][6].reshape(1, dp)
        wsrc = convs[i][0]
        if wsrc.shape[1] < 128:
            wsrc = jnp.pad(wsrc, ((0, 0), (0, 128 - wsrc.shape[1])))
        xs, sc2, hlin = _tc_mid(msg, den, hlin, b_prev,
                                wsrc, wscps[i], convs[i][7], dp, do)
        msg, den = _sc_edge(128, src, dst, es4[:, i], sc2[:, 0], sc2[:, 1], xs)
    return _tc_final(msg, den, hlin, convs[3][6].reshape(1, dos[3]), dos[3])


# speedup vs baseline: 26.1825x; 1.5075x over previous
import functools

import jax
import jax.numpy as jnp
from jax import lax
from jax.experimental import pallas as pl
from jax.experimental.pallas import tpu as pltpu
from jax.experimental.pallas import tpu_sc as plsc

N = 10000
E = 320000
NPAD = 10240
NTILE = 32
EPT = E // NTILE
CH = 80
R = 1024
ROWS_PER_TILE = NPAD // 16


def _f32(*shape):
    return jax.ShapeDtypeStruct(shape, jnp.float32)


_GDN = lax.GatherDimensionNumbers(
    offset_dims=(), collapsed_slice_dims=(0,), start_index_map=(0,))


def _bcast_lane(v16, lane):
    idx = jnp.full((16, 1), lane, jnp.int32)
    return lax.gather(v16, idx, _GDN, (1,),
                      mode=lax.GatherScatterMode.PROMISE_IN_BOUNDS)


def _pack_scores(sc):
    u = lax.bitcast_convert_type(sc, jnp.uint32)
    rne = u + jnp.uint32(0x7FFF) + ((u >> 16) & jnp.uint32(1))
    hi = rne & jnp.uint32(0xFFFF0000)
    lo = rne >> 16
    pk = hi | jnp.roll(lo, -1, axis=1)
    return lax.bitcast_convert_type(pk, jnp.int32)


def _tc_first_body(x_ref, wsrc_ref, wscp_ref, linw_ref, xs_ref, sc_ref, hlin_ref):
    h = x_ref[...]
    xs_ref[...] = jnp.dot(h, wsrc_ref[...], preferred_element_type=jnp.float32)
    sc_ref[...] = _pack_scores(
        jnp.dot(h, wscp_ref[...], preferred_element_type=jnp.float32))
    hlin_ref[...] = jnp.dot(h, linw_ref[...], preferred_element_type=jnp.float32)


def _finalize(msg_ref, den_ref, hlinp_ref, b_ref, dp):
    i = pl.program_id(0)
    m = (msg_ref[0] + msg_ref[1])[:, :dp]
    dn = jnp.sum(den_ref[:, pl.ds(i * R, R)], axis=0)
    return m / (dn[:, None] + 1e-30) + b_ref[...] + hlinp_ref[...]


def _tc_mid_body(dp, msg_ref, den_ref, hlinp_ref, b_ref, wsrc_ref, wscp_ref,
                 linw_ref, xs_ref, sc_ref, hlin_ref):
    h = jnp.maximum(_finalize(msg_ref, den_ref, hlinp_ref, b_ref, dp), 0.0)
    xs_ref[...] = jnp.dot(h, wsrc_ref[...], preferred_element_type=jnp.float32)
    sc_ref[...] = _pack_scores(
        jnp.dot(h, wscp_ref[...], preferred_element_type=jnp.float32))
    hlin_ref[...] = jnp.dot(h, linw_ref[...], preferred_element_type=jnp.float32)


def _tc_final_body(dp, msg_ref, den_ref, hlinp_ref, b_ref, out_ref):
    out_ref[...] = jax.nn.sigmoid(
        _finalize(msg_ref, den_ref, hlinp_ref, b_ref, dp))


def _tc_first(x, wsrc, wscp, linw, do):
    di = x.shape[1]
    return pl.pallas_call(
        _tc_first_body,
        grid=(pl.cdiv(N, R),),
        in_specs=[
            pl.BlockSpec((R, di), lambda i: (i, 0)),
            pl.BlockSpec((di, do), lambda i: (0, 0)),
            pl.BlockSpec((di, 128), lambda i: (0, 0)),
            pl.BlockSpec((di, do), lambda i: (0, 0)),
        ],
        out_specs=[
            pl.BlockSpec((R, do), lambda i: (i, 0)),
            pl.BlockSpec((R, 128), lambda i: (i, 0)),
            pl.BlockSpec((R, do), lambda i: (i, 0)),
        ],
        out_shape=[_f32(N, do),
                   jax.ShapeDtypeStruct((N, 128), jnp.int32),
                   _f32(N, do)],
    )(x, wsrc, wscp, linw)


def _tc_mid(msg, den, hlinp, b, wsrc, wscp, linw, dp, do):
    di = dp
    return pl.pallas_call(
        functools.partial(_tc_mid_body, dp),
        grid=(pl.cdiv(N, R),),
        in_specs=[
            pl.BlockSpec((2, R, 128), lambda i: (0, i, 0)),
            pl.BlockSpec((NTILE, NPAD), lambda i: (0, 0)),
            pl.BlockSpec((R, dp), lambda i: (i, 0)),
            pl.BlockSpec((1, dp), lambda i: (0, 0)),
            pl.BlockSpec((di, 128), lambda i: (0, 0)),
            pl.BlockSpec((di, 128), lambda i: (0, 0)),
            pl.BlockSpec((di, do), lambda i: (0, 0)),
        ],
        out_specs=[
            pl.BlockSpec((R, 128), lambda i: (i, 0)),
            pl.BlockSpec((R, 128), lambda i: (i, 0)),
            pl.BlockSpec((R, do), lambda i: (i, 0)),
        ],
        out_shape=[_f32(N, 128),
                   jax.ShapeDtypeStruct((N, 128), jnp.int32),
                   _f32(N, do)],
    )(msg, den, hlinp, b, wsrc, wscp, linw)


def _tc_final(msg, den, hlinp, b, dp):
    return pl.pallas_call(
        functools.partial(_tc_final_body, dp),
        grid=(pl.cdiv(N, R),),
        in_specs=[
            pl.BlockSpec((2, R, 128), lambda i: (0, i, 0)),
            pl.BlockSpec((NTILE, NPAD), lambda i: (0, 0)),
            pl.BlockSpec((R, dp), lambda i: (i, 0)),
            pl.BlockSpec((1, dp), lambda i: (0, 0)),
        ],
        out_specs=pl.BlockSpec((R, dp), lambda i: (i, 0)),
        out_shape=_f32(N, dp),
    )(msg, den, hlinp, b)


def _escore_body(ea_ref, bd_ref, out_ref):
    out_ref[...] = jnp.dot(ea_ref[...], bd_ref[...],
                           preferred_element_type=jnp.float32)


def _escore(ea2, bd):
    rows = ea2.shape[0]
    return pl.pallas_call(
        _escore_body,
        grid=(pl.cdiv(rows, R),),
        in_specs=[
            pl.BlockSpec((R, 128), lambda i: (i, 0)),
            pl.BlockSpec((128, 32), lambda i: (0, 0)),
        ],
        out_specs=pl.BlockSpec((R, 32), lambda i: (i, 0)),
        out_shape=_f32(rows, 32),
    )(ea2, bd)


T = EPT // CH
NBUF = 2


def _sc_body(ed_h, spk_h, xs_h, msg_o, den_o,
             spk_v, den_v, ec0, ec1, ex0, ex1, rows0, rows1, acc_s,
             es0, es1, gs0, gs1, ss0, ss1):
    cid = lax.axis_index("c")
    sid = lax.axis_index("s")
    w = sid * 2 + cid
    ecs = (ec0, ec1)
    exs = (ex0, ex1)
    rows = (rows0, rows1)
    esems = (es0, es1)
    gsems = (gs0, gs1)
    ssems = (ss0, ss1)

    pltpu.sync_copy(spk_h, spk_v)

    zero16 = jnp.zeros((16,), jnp.float32)

    def _zden(k, carry):
        den_v[pl.ds(k * 16, 16)] = zero16
        return carry
    lax.fori_loop(0, NPAD // 16, _zden, 0)

    def _zrow(k, carry):
        rr = k // 8
        cc = (k - rr * 8) * 16
        rows0[rr, pl.ds(cc, 16)] = zero16
        return carry
    lax.fori_loop(0, CH * 8, _zrow, 0)
    for k in range(ROWS_PER_TILE // CH):
        pltpu.sync_copy(rows0,
                        acc_s.at[pl.ds(sid * ROWS_PER_TILE + k * CH, CH)])
    plsc.subcore_barrier()

    def _ecopy_issue(t, b):
        pltpu.async_copy(ed_h.at[w, t], ecs[b], esems[b])

    def _ecopy_wait(b):
        pltpu.make_async_copy(ed_h.at[0, 0], ecs[b], esems[b]).wait()

    def _gather_issue(b):
        pltpu.async_copy(xs_h.at[ecs[b].at[0]], rows[b], gsems[b])

    def _gather_wait(b):
        pltpu.make_async_copy(xs_h.at[pl.ds(0, CH)], rows[b],
                              gsems[b]).wait()

    def _scatter_issue(b):
        pltpu.async_copy(rows[b], acc_s.at[ecs[b].at[1]], ssems[b],
                         add=True)

    def _scatter_wait(b):
        pltpu.make_async_copy(xs_h.at[pl.ds(0, CH)], rows[b],
                              ssems[b]).wait()

    hi_mask = jnp.full((16,), -65536, jnp.int32)

    def _scores(b):
        ec = ecs[b]
        exc = exs[b]
        for g in range(CH // 16):
            s16 = ec[0, pl.ds(g * 16, 16)]
            d16 = ec[1, pl.ds(g * 16, 16)]
            el16 = plsc.bitcast(ec[2, pl.ds(g * 16, 16)], jnp.float32)
            ws = plsc.load_gather(spk_v, [s16])
            wd = plsc.load_gather(spk_v, [d16])
            ssrc = plsc.bitcast(ws & hi_mask, jnp.float32)
            sdst = plsc.bitcast(wd << 16, jnp.float32)
            a = ssrc + sdst + el16
            a = jnp.where(a >= 0.0, a, 0.2 * a)
            ex = jnp.exp(a)
            exc[pl.ds(g * 16, 16)] = ex
            plsc.addupdate_scatter(den_v, [d16], ex)

    def _scale(b):
        rv = rows[b]
        exc = exs[b]

        def _sg(g, c2):
            ex16 = exc[pl.ds(g * 16, 16)]
            for l in range(16):
                exj = _bcast_lane(ex16, l)
                j = g * 16 + l
                for c in range(8):
                    rv[j, pl.ds(c * 16, 16)] = (
                        rv[j, pl.ds(c * 16, 16)] * exj)
            return c2
        lax.fori_loop(0, CH // 16, _sg, 0)

    def _pipe(t, b):
        nb = (b + 1) % NBUF
        _ecopy_wait(b)
        _gather_issue(b)
        _scores(b)

        @pl.when(t >= 1)
        def _():
            _scatter_wait(nb)

        @pl.when(t + 1 < T)
        def _():
            _ecopy_issue(t + 1, nb)
        _gather_wait(b)
        _scale(b)
        _scatter_issue(b)

    _ecopy_issue(0, 0)
    _pipe(0, 0)

    def _pair(k, carry):
        t0 = 1 + k * 2
        for j in range(2):
            _pipe(t0 + j, (1 + j) % NBUF)
        return carry
    lax.fori_loop(0, (T - 1) // 2, _pair, 0)
    _scatter_wait((T - 1) % NBUF)

    plsc.subcore_barrier()
    pltpu.sync_copy(acc_s.at[pl.ds(sid * ROWS_PER_TILE, ROWS_PER_TILE)],
                    msg_o.at[cid, pl.ds(sid * ROWS_PER_TILE, ROWS_PER_TILE)])
    pltpu.sync_copy(den_v, den_o.at[w])


def _sc_edge(ed4, spk, xs):
    mesh = plsc.VectorSubcoreMesh(core_axis_name="c", subcore_axis_name="s")
    kern = pl.kernel(
        _sc_body,
        mesh=mesh,
        compiler_params=pltpu.CompilerParams(needs_layout_passes=False),
        out_type=[_f32(2, NPAD, 128), _f32(NTILE, NPAD)],
        scratch_types=[
            pltpu.VMEM((N,), jnp.int32),
            pltpu.VMEM((NPAD,), jnp.float32),
            pltpu.VMEM((3, CH), jnp.int32),
            pltpu.VMEM((3, CH), jnp.int32),
            pltpu.VMEM((CH,), jnp.float32),
            pltpu.VMEM((CH,), jnp.float32),
            pltpu.VMEM((CH, 128), jnp.float32),
            pltpu.VMEM((CH, 128), jnp.float32),
            pltpu.VMEM_SHARED((NPAD, 128), jnp.float32),
            pltpu.SemaphoreType.DMA,
            pltpu.SemaphoreType.DMA,
            pltpu.SemaphoreType.DMA,
            pltpu.SemaphoreType.DMA,
            pltpu.SemaphoreType.DMA,
            pltpu.SemaphoreType.DMA,
        ],
    )
    return kern(ed4, spk, xs)


def kernel(x, edge_index, edge_attr,
           conv1_Wsrc, conv1_Wdst, conv1_We, conv1_asrc, conv1_adst, conv1_ae, conv1_b, lin1_W,
           conv2_Wsrc, conv2_Wdst, conv2_We, conv2_asrc, conv2_adst, conv2_ae, conv2_b, lin2_W,
           conv3_Wsrc, conv3_Wdst, conv3_We, conv3_asrc, conv3_adst, conv3_ae, conv3_b, lin3_W,
           conv4_Wsrc, conv4_Wdst, conv4_We, conv4_asrc, conv4_adst, conv4_ae, conv4_b, lin4_W):
    convs = [
        (conv1_Wsrc, conv1_Wdst, conv1_We, conv1_asrc, conv1_adst, conv1_ae, conv1_b, lin1_W),
        (conv2_Wsrc, conv2_Wdst, conv2_We, conv2_asrc, conv2_adst, conv2_ae, conv2_b, lin2_W),
        (conv3_Wsrc, conv3_Wdst, conv3_We, conv3_asrc, conv3_adst, conv3_ae, conv3_b, lin3_W),
        (conv4_Wsrc, conv4_Wdst, conv4_We, conv4_asrc, conv4_adst, conv4_ae, conv4_b, lin4_W),
    ]
    srcp = edge_index[0].reshape(NTILE, T, 1, CH)
    dstp = edge_index[1].reshape(NTILE, T, 1, CH)

    wscps = []
    for (Wsrc, Wdst, We, asrc, adst, ae, b, linW) in convs:
        wsa = Wsrc @ asrc
        wdd = Wdst @ adst
        wscp = jnp.zeros((Wsrc.shape[0], 128), jnp.float32)
        wscp = wscp.at[:, 0].set(wsa).at[:, 1].set(wdd)
        wscps.append(wscp)
    blk = jnp.stack([c[2] @ c[5] for c in convs], axis=1)
    bd = jnp.kron(jnp.eye(8, dtype=jnp.float32), blk)

    es4 = _escore(edge_attr.reshape(E // 8, 128), bd).reshape(E, 4)

    dos = [128, 128, 128, 64]
    xs, sc2, hlin = _tc_first(x, convs[0][0], wscps[0], convs[0][7], 128)
    ed4 = jnp.concatenate(
        [srcp, dstp,
         lax.bitcast_convert_type(es4[:, 0], jnp.int32).reshape(
             NTILE, T, 1, CH)], axis=2)
    msg, den = _sc_edge(ed4, sc2[:, 0], xs)
    for i in (1, 2, 3):
        dp, do = dos[i - 1], dos[i]
        b_prev = convs[i - 1][6].reshape(1, dp)
        wsrc = convs[i][0]
        if wsrc.shape[1] < 128:
            wsrc = jnp.pad(wsrc, ((0, 0), (0, 128 - wsrc.shape[1])))
        xs, sc2, hlin = _tc_mid(msg, den, hlin, b_prev,
                                wsrc, wscps[i], convs[i][7], dp, do)
        ed4 = jnp.concatenate(
            [srcp, dstp,
             lax.bitcast_convert_type(es4[:, i], jnp.int32).reshape(
                 NTILE, T, 1, CH)], axis=2)
        msg, den = _sc_edge(ed4, sc2[:, 0], xs)
    return _tc_final(msg, den, hlin, convs[3][6].reshape(1, dos[3]), dos[3])


# speedup vs baseline: 31.0081x; 1.1843x over previous
import functools

import jax
import jax.numpy as jnp
from jax import lax
from jax.experimental import pallas as pl
from jax.experimental.pallas import tpu as pltpu
from jax.experimental.pallas import tpu_sc as plsc

N = 10000
E = 320000
NPAD = 10240
NTILE = 32
EPT = E // NTILE
CH = 80
R = 1024
ROWS_PER_TILE = NPAD // 16


def _f32(*shape):
    return jax.ShapeDtypeStruct(shape, jnp.float32)


_GDN = lax.GatherDimensionNumbers(
    offset_dims=(), collapsed_slice_dims=(0,), start_index_map=(0,))


def _bcast_lane(v16, lane):
    idx = jnp.full((16, 1), lane, jnp.int32)
    return lax.gather(v16, idx, _GDN, (1,),
                      mode=lax.GatherScatterMode.PROMISE_IN_BOUNDS)


def _pack_scores(sc):
    u = lax.bitcast_convert_type(sc, jnp.uint32)
    rne = u + jnp.uint32(0x7FFF) + ((u >> 16) & jnp.uint32(1))
    hi = rne & jnp.uint32(0xFFFF0000)
    lo = rne >> 16
    pk = hi | jnp.roll(lo, -1, axis=1)
    return lax.bitcast_convert_type(pk, jnp.int32)


def _tc_first_body(x_ref, wsrc_ref, wscp_ref, linw_ref, xs_ref, sc_ref, hlin_ref):
    h = x_ref[...]
    xs_ref[...] = jnp.dot(h, wsrc_ref[...], preferred_element_type=jnp.float32)
    sc_ref[...] = _pack_scores(
        jnp.dot(h, wscp_ref[...], preferred_element_type=jnp.float32))
    hlin_ref[...] = jnp.dot(h, linw_ref[...], preferred_element_type=jnp.float32)


def _finalize(msg_ref, den_ref, hlinp_ref, b_ref, dp):
    m = (msg_ref[0] + msg_ref[1])[:, :dp]
    dn = jnp.sum(den_ref[...], axis=0)
    return m / (dn[:, None] + 1e-30) + b_ref[...] + hlinp_ref[...]


def _tc_mid_body(dp, msg_ref, den_ref, hlinp_ref, b_ref, wsrc_ref, wscp_ref,
                 linw_ref, xs_ref, sc_ref, hlin_ref):
    h = jnp.maximum(_finalize(msg_ref, den_ref, hlinp_ref, b_ref, dp), 0.0)
    xs_ref[...] = jnp.dot(h, wsrc_ref[...], preferred_element_type=jnp.float32)
    sc_ref[...] = _pack_scores(
        jnp.dot(h, wscp_ref[...], preferred_element_type=jnp.float32))
    hlin_ref[...] = jnp.dot(h, linw_ref[...], preferred_element_type=jnp.float32)


def _tc_final_body(dp, msg_ref, den_ref, hlinp_ref, b_ref, out_ref):
    out_ref[...] = jax.nn.sigmoid(
        _finalize(msg_ref, den_ref, hlinp_ref, b_ref, dp))


def _tc_first(x, wsrc, wscp, linw, do):
    di = x.shape[1]
    return pl.pallas_call(
        _tc_first_body,
        grid=(pl.cdiv(N, R),),
        in_specs=[
            pl.BlockSpec((R, di), lambda i: (i, 0)),
            pl.BlockSpec((di, do), lambda i: (0, 0)),
            pl.BlockSpec((di, 128), lambda i: (0, 0)),
            pl.BlockSpec((di, do), lambda i: (0, 0)),
        ],
        out_specs=[
            pl.BlockSpec((R, do), lambda i: (i, 0)),
            pl.BlockSpec((R, 128), lambda i: (i, 0)),
            pl.BlockSpec((R, do), lambda i: (i, 0)),
        ],
        out_shape=[_f32(N, do),
                   jax.ShapeDtypeStruct((N, 128), jnp.int32),
                   _f32(N, do)],
    )(x, wsrc, wscp, linw)


def _tc_mid(msg, den, hlinp, b, wsrc, wscp, linw, dp, do):
    di = dp
    return pl.pallas_call(
        functools.partial(_tc_mid_body, dp),
        grid=(pl.cdiv(N, R),),
        in_specs=[
            pl.BlockSpec((2, R, 128), lambda i: (0, i, 0)),
            pl.BlockSpec((NTILE, R), lambda i: (0, i)),
            pl.BlockSpec((R, dp), lambda i: (i, 0)),
            pl.BlockSpec((1, dp), lambda i: (0, 0)),
            pl.BlockSpec((di, 128), lambda i: (0, 0)),
            pl.BlockSpec((di, 128), lambda i: (0, 0)),
            pl.BlockSpec((di, do), lambda i: (0, 0)),
        ],
        out_specs=[
            pl.BlockSpec((R, 128), lambda i: (i, 0)),
            pl.BlockSpec((R, 128), lambda i: (i, 0)),
            pl.BlockSpec((R, do), lambda i: (i, 0)),
        ],
        out_shape=[_f32(N, 128),
                   jax.ShapeDtypeStruct((N, 128), jnp.int32),
                   _f32(N, do)],
    )(msg, den, hlinp, b, wsrc, wscp, linw)


def _tc_final(msg, den, hlinp, b, dp):
    return pl.pallas_call(
        functools.partial(_tc_final_body, dp),
        grid=(pl.cdiv(N, R),),
        in_specs=[
            pl.BlockSpec((2, R, 128), lambda i: (0, i, 0)),
            pl.BlockSpec((NTILE, R), lambda i: (0, i)),
            pl.BlockSpec((R, dp), lambda i: (i, 0)),
            pl.BlockSpec((1, dp), lambda i: (0, 0)),
        ],
        out_specs=pl.BlockSpec((R, dp), lambda i: (i, 0)),
        out_shape=_f32(N, dp),
    )(msg, den, hlinp, b)


def _escore_body(ea_ref, bd_ref, out_ref):
    out_ref[...] = jnp.dot(ea_ref[...], bd_ref[...],
                           preferred_element_type=jnp.float32)


def _escore(ea2, bd):
    rows = ea2.shape[0]
    return pl.pallas_call(
        _escore_body,
        grid=(pl.cdiv(rows, R),),
        in_specs=[
            pl.BlockSpec((R, 128), lambda i: (i, 0)),
            pl.BlockSpec((128, 32), lambda i: (0, 0)),
        ],
        out_specs=pl.BlockSpec((R, 32), lambda i: (i, 0)),
        out_shape=_f32(rows, 32),
    )(ea2, bd)


T = EPT // CH
NBUF = 2


def _sc_body(li, ed_h, spk_h, xs_h, msg_o, den_o,
             spk_v, den_v, ec0, ec1, ex0, ex1, rows0, rows1, acc_s,
             es0, es1, gs0, gs1, ss0, ss1):
    cid = lax.axis_index("c")
    sid = lax.axis_index("s")
    w = sid * 2 + cid
    ecs = (ec0, ec1)
    exs = (ex0, ex1)
    rows = (rows0, rows1)
    esems = (es0, es1)
    gsems = (gs0, gs1)
    ssems = (ss0, ss1)

    pltpu.sync_copy(spk_h, spk_v)

    zero16 = jnp.zeros((16,), jnp.float32)

    def _zden(k, carry):
        den_v[pl.ds(k * 16, 16)] = zero16
        return carry
    lax.fori_loop(0, NPAD // 16, _zden, 0)

    def _zrow(k, carry):
        rr = k // 8
        cc = (k - rr * 8) * 16
        rows0[rr, pl.ds(cc, 16)] = zero16
        return carry
    lax.fori_loop(0, CH * 8, _zrow, 0)
    for k in range(ROWS_PER_TILE // CH):
        pltpu.sync_copy(rows0,
                        acc_s.at[pl.ds(sid * ROWS_PER_TILE + k * CH, CH)])
    plsc.subcore_barrier()

    def _ecopy_issue(t, b):
        pltpu.async_copy(ed_h.at[w, t], ecs[b], esems[b])

    def _ecopy_wait(b):
        pltpu.make_async_copy(ed_h.at[0, 0], ecs[b], esems[b]).wait()

    def _gather_issue(b):
        pltpu.async_copy(xs_h.at[ecs[b].at[0]], rows[b], gsems[b])

    def _gather_wait(b):
        pltpu.make_async_copy(xs_h.at[pl.ds(0, CH)], rows[b],
                              gsems[b]).wait()

    def _scatter_issue(b):
        pltpu.async_copy(rows[b], acc_s.at[ecs[b].at[1]], ssems[b],
                         add=True)

    def _scatter_wait(b):
        pltpu.make_async_copy(xs_h.at[pl.ds(0, CH)], rows[b],
                              ssems[b]).wait()

    hi_mask = jnp.full((16,), -65536, jnp.int32)

    def _scores(b):
        ec = ecs[b]
        exc = exs[b]
        for g in range(CH // 16):
            s16 = ec[0, pl.ds(g * 16, 16)]
            d16 = ec[1, pl.ds(g * 16, 16)]
            el16 = plsc.bitcast(ec[2 + li, pl.ds(g * 16, 16)], jnp.float32)
            ws = plsc.load_gather(spk_v, [s16])
            wd = plsc.load_gather(spk_v, [d16])
            ssrc = plsc.bitcast(ws & hi_mask, jnp.float32)
            sdst = plsc.bitcast(wd << 16, jnp.float32)
            a = ssrc + sdst + el16
            a = jnp.where(a >= 0.0, a, 0.2 * a)
            ex = jnp.exp(a)
            exc[pl.ds(g * 16, 16)] = ex
            plsc.addupdate_scatter(den_v, [d16], ex)

    def _scale(b):
        rv = rows[b]
        exc = exs[b]

        def _sg(g, c2):
            ex16 = exc[pl.ds(g * 16, 16)]
            for l in range(16):
                exj = _bcast_lane(ex16, l)
                j = g * 16 + l
                for c in range(8):
                    rv[j, pl.ds(c * 16, 16)] = (
                        rv[j, pl.ds(c * 16, 16)] * exj)
            return c2
        lax.fori_loop(0, CH // 16, _sg, 0)

    def _pipe(t, b):
        nb = (b + 1) % NBUF
        _ecopy_wait(b)
        _gather_issue(b)
        _scores(b)

        @pl.when(t >= 1)
        def _():
            _scatter_wait(nb)

        @pl.when(t + 1 < T)
        def _():
            _ecopy_issue(t + 1, nb)
        _gather_wait(b)
        _scale(b)
        _scatter_issue(b)

    _ecopy_issue(0, 0)
    _pipe(0, 0)

    def _pair(k, carry):
        t0 = 1 + k * 2
        for j in range(2):
            _pipe(t0 + j, (1 + j) % NBUF)
        return carry
    lax.fori_loop(0, (T - 1) // 2, _pair, 0)
    _scatter_wait((T - 1) % NBUF)

    plsc.subcore_barrier()
    pltpu.sync_copy(acc_s.at[pl.ds(sid * ROWS_PER_TILE, ROWS_PER_TILE)],
                    msg_o.at[cid, pl.ds(sid * ROWS_PER_TILE, ROWS_PER_TILE)])
    pltpu.sync_copy(den_v, den_o.at[w])


def _sc_edge(li, ed5, spk, xs):
    mesh = plsc.VectorSubcoreMesh(core_axis_name="c", subcore_axis_name="s")
    kern = pl.kernel(
        functools.partial(_sc_body, li),
        mesh=mesh,
        compiler_params=pltpu.CompilerParams(needs_layout_passes=False),
        out_type=[_f32(2, NPAD, 128), _f32(NTILE, NPAD)],
        scratch_types=[
            pltpu.VMEM((N,), jnp.int32),
            pltpu.VMEM((NPAD,), jnp.float32),
            pltpu.VMEM((6, CH), jnp.int32),
            pltpu.VMEM((6, CH), jnp.int32),
            pltpu.VMEM((CH,), jnp.float32),
            pltpu.VMEM((CH,), jnp.float32),
            pltpu.VMEM((CH, 128), jnp.float32),
            pltpu.VMEM((CH, 128), jnp.float32),
            pltpu.VMEM_SHARED((NPAD, 128), jnp.float32),
            pltpu.SemaphoreType.DMA,
            pltpu.SemaphoreType.DMA,
            pltpu.SemaphoreType.DMA,
            pltpu.SemaphoreType.DMA,
            pltpu.SemaphoreType.DMA,
            pltpu.SemaphoreType.DMA,
        ],
    )
    return kern(ed5, spk, xs)


def kernel(x, edge_index, edge_attr,
           conv1_Wsrc, conv1_Wdst, conv1_We, conv1_asrc, conv1_adst, conv1_ae, conv1_b, lin1_W,
           conv2_Wsrc, conv2_Wdst, conv2_We, conv2_asrc, conv2_adst, conv2_ae, conv2_b, lin2_W,
           conv3_Wsrc, conv3_Wdst, conv3_We, conv3_asrc, conv3_adst, conv3_ae, conv3_b, lin3_W,
           conv4_Wsrc, conv4_Wdst, conv4_We, conv4_asrc, conv4_adst, conv4_ae, conv4_b, lin4_W):
    convs = [
        (conv1_Wsrc, conv1_Wdst, conv1_We, conv1_asrc, conv1_adst, conv1_ae, conv1_b, lin1_W),
        (conv2_Wsrc, conv2_Wdst, conv2_We, conv2_asrc, conv2_adst, conv2_ae, conv2_b, lin2_W),
        (conv3_Wsrc, conv3_Wdst, conv3_We, conv3_asrc, conv3_adst, conv3_ae, conv3_b, lin3_W),
        (conv4_Wsrc, conv4_Wdst, conv4_We, conv4_asrc, conv4_adst, conv4_ae, conv4_b, lin4_W),
    ]
    srcp = edge_index[0].reshape(NTILE, T, 1, CH)
    dstp = edge_index[1].reshape(NTILE, T, 1, CH)

    wscps = []
    for (Wsrc, Wdst, We, asrc, adst, ae, b, linW) in convs:
        wsa = Wsrc @ asrc
        wdd = Wdst @ adst
        wscp = jnp.zeros((Wsrc.shape[0], 128), jnp.float32)
        wscp = wscp.at[:, 0].set(wsa).at[:, 1].set(wdd)
        wscps.append(wscp)
    blk = jnp.stack([c[2] @ c[5] for c in convs], axis=1)
    bd = jnp.kron(jnp.eye(8, dtype=jnp.float32), blk)

    es4 = _escore(edge_attr.reshape(E // 8, 128), bd).reshape(E, 4)

    dos = [128, 128, 128, 64]
    xs, sc2, hlin = _tc_first(x, convs[0][0], wscps[0], convs[0][7], 128)
    el4t = lax.bitcast_convert_type(es4, jnp.int32).reshape(
        NTILE, T, CH, 4).transpose(0, 1, 3, 2)
    ed5 = jnp.concatenate([srcp, dstp, el4t], axis=2)
    msg, den = _sc_edge(0, ed5, sc2[:, 0], xs)
    for i in (1, 2, 3):
        dp, do = dos[i - 1], dos[i]
        b_prev = convs[i - 1][6].reshape(1, dp)
        wsrc = convs[i][0]
        if wsrc.shape[1] < 128:
            wsrc = jnp.pad(wsrc, ((0, 0), (0, 128 - wsrc.shape[1])))
        xs, sc2, hlin = _tc_mid(msg, den, hlin, b_prev,
                                wsrc, wscps[i], convs[i][7], dp, do)
        msg, den = _sc_edge(i, ed5, sc2[:, 0], xs)
    return _tc_final(msg, den, hlin, convs[3][6].reshape(1, dos[3]), dos[3])


# speedup vs baseline: 31.0593x; 1.0017x over previous
import functools

import jax
import jax.numpy as jnp
from jax import lax
from jax.experimental import pallas as pl
from jax.experimental.pallas import tpu as pltpu
from jax.experimental.pallas import tpu_sc as plsc

N = 10000
E = 320000
NPAD = 10240
NTILE = 32
EPT = E // NTILE
CH = 80
R = 1024
ROWS_PER_TILE = NPAD // 16


def _f32(*shape):
    return jax.ShapeDtypeStruct(shape, jnp.float32)


_GDN = lax.GatherDimensionNumbers(
    offset_dims=(), collapsed_slice_dims=(0,), start_index_map=(0,))


def _bcast_lane(v16, lane):
    idx = jnp.full((16, 1), lane, jnp.int32)
    return lax.gather(v16, idx, _GDN, (1,),
                      mode=lax.GatherScatterMode.PROMISE_IN_BOUNDS)


def _pack_scores(sc):
    u = lax.bitcast_convert_type(sc, jnp.uint32)
    rne = u + jnp.uint32(0x7FFF) + ((u >> 16) & jnp.uint32(1))
    hi = rne & jnp.uint32(0xFFFF0000)
    lo = rne >> 16
    pk = hi | jnp.roll(lo, -1, axis=1)
    return lax.bitcast_convert_type(pk, jnp.int32)[:, :8]


def _tc_first_body(x_ref, wsrc_ref, wscp_ref, linw_ref, xs_ref, sc_ref, hlin_ref):
    h = x_ref[...]
    xs_ref[...] = jnp.dot(h, wsrc_ref[...], preferred_element_type=jnp.float32)
    sc_ref[...] = _pack_scores(
        jnp.dot(h, wscp_ref[...], preferred_element_type=jnp.float32))
    hlin_ref[...] = jnp.dot(h, linw_ref[...], preferred_element_type=jnp.float32)


def _finalize(msg_ref, den_ref, hlinp_ref, b_ref, dp):
    m = (msg_ref[0] + msg_ref[1])[:, :dp]
    dn = jnp.sum(den_ref[...], axis=0)
    return m / (dn[:, None] + 1e-30) + b_ref[...] + hlinp_ref[...]


def _tc_mid_body(dp, msg_ref, den_ref, hlinp_ref, b_ref, wsrc_ref, wscp_ref,
                 linw_ref, xs_ref, sc_ref, hlin_ref):
    h = jnp.maximum(_finalize(msg_ref, den_ref, hlinp_ref, b_ref, dp), 0.0)
    xs_ref[...] = jnp.dot(h, wsrc_ref[...], preferred_element_type=jnp.float32)
    sc_ref[...] = _pack_scores(
        jnp.dot(h, wscp_ref[...], preferred_element_type=jnp.float32))
    hlin_ref[...] = jnp.dot(h, linw_ref[...], preferred_element_type=jnp.float32)


def _tc_final_body(dp, msg_ref, den_ref, hlinp_ref, b_ref, out_ref):
    out_ref[...] = jax.nn.sigmoid(
        _finalize(msg_ref, den_ref, hlinp_ref, b_ref, dp))


def _tc_first(x, wsrc, wscp, linw, do):
    di = x.shape[1]
    return pl.pallas_call(
        _tc_first_body,
        grid=(pl.cdiv(N, R),),
        in_specs=[
            pl.BlockSpec((R, di), lambda i: (i, 0)),
            pl.BlockSpec((di, do), lambda i: (0, 0)),
            pl.BlockSpec((di, 128), lambda i: (0, 0)),
            pl.BlockSpec((di, do), lambda i: (0, 0)),
        ],
        out_specs=[
            pl.BlockSpec((R, do), lambda i: (i, 0)),
            pl.BlockSpec((R, 8), lambda i: (i, 0)),
            pl.BlockSpec((R, do), lambda i: (i, 0)),
        ],
        out_shape=[_f32(N, do),
                   jax.ShapeDtypeStruct((N, 8), jnp.int32),
                   _f32(N, do)],
    )(x, wsrc, wscp, linw)


def _tc_mid(msg, den, hlinp, b, wsrc, wscp, linw, dp, do):
    di = dp
    return pl.pallas_call(
        functools.partial(_tc_mid_body, dp),
        grid=(pl.cdiv(N, R),),
        in_specs=[
            pl.BlockSpec((2, R, 128), lambda i: (0, i, 0)),
            pl.BlockSpec((NTILE, R), lambda i: (0, i)),
            pl.BlockSpec((R, dp), lambda i: (i, 0)),
            pl.BlockSpec((1, dp), lambda i: (0, 0)),
            pl.BlockSpec((di, 128), lambda i: (0, 0)),
            pl.BlockSpec((di, 128), lambda i: (0, 0)),
            pl.BlockSpec((di, do), lambda i: (0, 0)),
        ],
        out_specs=[
            pl.BlockSpec((R, 128), lambda i: (i, 0)),
            pl.BlockSpec((R, 8), lambda i: (i, 0)),
            pl.BlockSpec((R, do), lambda i: (i, 0)),
        ],
        out_shape=[_f32(N, 128),
                   jax.ShapeDtypeStruct((N, 8), jnp.int32),
                   _f32(N, do)],
    )(msg, den, hlinp, b, wsrc, wscp, linw)


def _tc_final(msg, den, hlinp, b, dp):
    return pl.pallas_call(
        functools.partial(_tc_final_body, dp),
        grid=(pl.cdiv(N, R),),
        in_specs=[
            pl.BlockSpec((2, R, 128), lambda i: (0, i, 0)),
            pl.BlockSpec((NTILE, R), lambda i: (0, i)),
            pl.BlockSpec((R, dp), lambda i: (i, 0)),
            pl.BlockSpec((1, dp), lambda i: (0, 0)),
        ],
        out_specs=pl.BlockSpec((R, dp), lambda i: (i, 0)),
        out_shape=_f32(N, dp),
    )(msg, den, hlinp, b)


def _escore_body(ea_ref, bd_ref, out_ref):
    out_ref[...] = jnp.dot(ea_ref[...], bd_ref[...],
                           preferred_element_type=jnp.float32)


def _escore(ea2, bd):
    rows = ea2.shape[0]
    return pl.pallas_call(
        _escore_body,
        grid=(pl.cdiv(rows, R),),
        in_specs=[
            pl.BlockSpec((R, 128), lambda i: (i, 0)),
            pl.BlockSpec((128, 32), lambda i: (0, 0)),
        ],
        out_specs=pl.BlockSpec((R, 32), lambda i: (i, 0)),
        out_shape=_f32(rows, 32),
    )(ea2, bd)


T = EPT // CH
NBUF = 2


def _sc_body(li, ed_h, spk_h, xs_h, msg_o, den_o,
             spk_v, den_v, ec0, ec1, ex0, ex1, rows0, rows1, acc_s,
             es0, es1, gs0, gs1, ss0, ss1):
    cid = lax.axis_index("c")
    sid = lax.axis_index("s")
    w = sid * 2 + cid
    ecs = (ec0, ec1)
    exs = (ex0, ex1)
    rows = (rows0, rows1)
    esems = (es0, es1)
    gsems = (gs0, gs1)
    ssems = (ss0, ss1)

    pltpu.sync_copy(spk_h, spk_v)

    zero16 = jnp.zeros((16,), jnp.float32)

    def _zden(k, carry):
        den_v[pl.ds(k * 16, 16)] = zero16
        return carry
    lax.fori_loop(0, NPAD // 16, _zden, 0)

    def _zrow(k, carry):
        rr = k // 8
        cc = (k - rr * 8) * 16
        rows0[rr, pl.ds(cc, 16)] = zero16
        return carry
    lax.fori_loop(0, CH * 8, _zrow, 0)
    for k in range(ROWS_PER_TILE // CH):
        pltpu.async_copy(rows0,
                         acc_s.at[pl.ds(sid * ROWS_PER_TILE + k * CH, CH)],
                         gs0)
    for k in range(ROWS_PER_TILE // CH):
        pltpu.make_async_copy(
            rows0, acc_s.at[pl.ds(sid * ROWS_PER_TILE, CH)], gs0).wait()
    plsc.subcore_barrier()

    def _ecopy_issue(t, b):
        pltpu.async_copy(ed_h.at[w, t], ecs[b], esems[b])

    def _ecopy_wait(b):
        pltpu.make_async_copy(ed_h.at[0, 0], ecs[b], esems[b]).wait()

    def _gather_issue(b):
        pltpu.async_copy(xs_h.at[ecs[b].at[0]], rows[b], gsems[b])

    def _gather_wait(b):
        pltpu.make_async_copy(xs_h.at[pl.ds(0, CH)], rows[b],
                              gsems[b]).wait()

    def _scatter_issue(b):
        pltpu.async_copy(rows[b], acc_s.at[ecs[b].at[1]], ssems[b],
                         add=True)

    def _scatter_wait(b):
        pltpu.make_async_copy(xs_h.at[pl.ds(0, CH)], rows[b],
                              ssems[b]).wait()

    hi_mask = jnp.full((16,), -65536, jnp.int32)

    def _scores(b):
        ec = ecs[b]
        exc = exs[b]
        for g in range(CH // 16):
            s16 = ec[0, pl.ds(g * 16, 16)]
            d16 = ec[1, pl.ds(g * 16, 16)]
            el16 = plsc.bitcast(ec[2 + li, pl.ds(g * 16, 16)], jnp.float32)
            ws = plsc.load_gather(spk_v, [s16])
            wd = plsc.load_gather(spk_v, [d16])
            ssrc = plsc.bitcast(ws & hi_mask, jnp.float32)
            sdst = plsc.bitcast(wd << 16, jnp.float32)
            a = ssrc + sdst + el16
            a = jnp.where(a >= 0.0, a, 0.2 * a)
            ex = jnp.exp(a)
            exc[pl.ds(g * 16, 16)] = ex
            plsc.addupdate_scatter(den_v, [d16], ex)

    def _scale(b):
        rv = rows[b]
        exc = exs[b]

        def _sg(g, c2):
            ex16 = exc[pl.ds(g * 16, 16)]
            for l in range(16):
                exj = _bcast_lane(ex16, l)
                j = g * 16 + l
                for c in range(8):
                    rv[j, pl.ds(c * 16, 16)] = (
                        rv[j, pl.ds(c * 16, 16)] * exj)
            return c2
        lax.fori_loop(0, CH // 16, _sg, 0)

    def _pipe(t, b):
        nb = (b + 1) % NBUF
        _ecopy_wait(b)
        _gather_issue(b)
        _scores(b)

        @pl.when(t >= 1)
        def _():
            _scatter_wait(nb)

        @pl.when(t + 1 < T)
        def _():
            _ecopy_issue(t + 1, nb)
        _gather_wait(b)
        _scale(b)
        _scatter_issue(b)

    _ecopy_issue(0, 0)
    _pipe(0, 0)

    def _pair(k, carry):
        t0 = 1 + k * 2
        for j in range(2):
            _pipe(t0 + j, (1 + j) % NBUF)
        return carry
    lax.fori_loop(0, (T - 1) // 2, _pair, 0)
    _scatter_wait((T - 1) % NBUF)

    plsc.subcore_barrier()
    pltpu.sync_copy(acc_s.at[pl.ds(sid * ROWS_PER_TILE, ROWS_PER_TILE)],
                    msg_o.at[cid, pl.ds(sid * ROWS_PER_TILE, ROWS_PER_TILE)])
    pltpu.sync_copy(den_v, den_o.at[w])


def _sc_edge(li, ed5, spk, xs):
    mesh = plsc.VectorSubcoreMesh(core_axis_name="c", subcore_axis_name="s")
    kern = pl.kernel(
        functools.partial(_sc_body, li),
        mesh=mesh,
        compiler_params=pltpu.CompilerParams(needs_layout_passes=False),
        out_type=[_f32(2, NPAD, 128), _f32(NTILE, NPAD)],
        scratch_types=[
            pltpu.VMEM((N,), jnp.int32),
            pltpu.VMEM((NPAD,), jnp.float32),
            pltpu.VMEM((6, CH), jnp.int32),
            pltpu.VMEM((6, CH), jnp.int32),
            pltpu.VMEM((CH,), jnp.float32),
            pltpu.VMEM((CH,), jnp.float32),
            pltpu.VMEM((CH, 128), jnp.float32),
            pltpu.VMEM((CH, 128), jnp.float32),
            pltpu.VMEM_SHARED((NPAD, 128), jnp.float32),
            pltpu.SemaphoreType.DMA,
            pltpu.SemaphoreType.DMA,
            pltpu.SemaphoreType.DMA,
            pltpu.SemaphoreType.DMA,
            pltpu.SemaphoreType.DMA,
            pltpu.SemaphoreType.DMA,
        ],
    )
    return kern(ed5, spk, xs)


def kernel(x, edge_index, edge_attr,
           conv1_Wsrc, conv1_Wdst, conv1_We, conv1_asrc, conv1_adst, conv1_ae, conv1_b, lin1_W,
           conv2_Wsrc, conv2_Wdst, conv2_We, conv2_asrc, conv2_adst, conv2_ae, conv2_b, lin2_W,
           conv3_Wsrc, conv3_Wdst, conv3_We, conv3_asrc, conv3_adst, conv3_ae, conv3_b, lin3_W,
           conv4_Wsrc, conv4_Wdst, conv4_We, conv4_asrc, conv4_adst, conv4_ae, conv4_b, lin4_W):
    convs = [
        (conv1_Wsrc, conv1_Wdst, conv1_We, conv1_asrc, conv1_adst, conv1_ae, conv1_b, lin1_W),
        (conv2_Wsrc, conv2_Wdst, conv2_We, conv2_asrc, conv2_adst, conv2_ae, conv2_b, lin2_W),
        (conv3_Wsrc, conv3_Wdst, conv3_We, conv3_asrc, conv3_adst, conv3_ae, conv3_b, lin3_W),
        (conv4_Wsrc, conv4_Wdst, conv4_We, conv4_asrc, conv4_adst, conv4_ae, conv4_b, lin4_W),
    ]
    srcp = edge_index[0].reshape(NTILE, T, 1, CH)
    dstp = edge_index[1].reshape(NTILE, T, 1, CH)

    wscps = []
    for (Wsrc, Wdst, We, asrc, adst, ae, b, linW) in convs:
        wsa = Wsrc @ asrc
        wdd = Wdst @ adst
        wscp = jnp.zeros((Wsrc.shape[0], 128), jnp.float32)
        wscp = wscp.at[:, 0].set(wsa).at[:, 1].set(wdd)
        wscps.append(wscp)
    blk = jnp.stack([c[2] @ c[5] for c in convs], axis=1)
    bd = jnp.kron(jnp.eye(8, dtype=jnp.float32), blk)

    es4 = _escore(edge_attr.reshape(E // 8, 128), bd).reshape(E, 4)

    dos = [128, 128, 128, 64]
    xs, sc2, hlin = _tc_first(x, convs[0][0], wscps[0], convs[0][7], 128)
    el4t = lax.bitcast_convert_type(es4, jnp.int32).reshape(
        NTILE, T, CH, 4).transpose(0, 1, 3, 2)
    ed5 = jnp.concatenate([srcp, dstp, el4t], axis=2)
    msg, den = _sc_edge(0, ed5, sc2[:, 0], xs)
    for i in (1, 2, 3):
        dp, do = dos[i - 1], dos[i]
        b_prev = convs[i - 1][6].reshape(1, dp)
        wsrc = convs[i][0]
        if wsrc.shape[1] < 128:
            wsrc = jnp.pad(wsrc, ((0, 0), (0, 128 - wsrc.shape[1])))
        xs, sc2, hlin = _tc_mid(msg, den, hlin, b_prev,
                                wsrc, wscps[i], convs[i][7], dp, do)
        msg, den = _sc_edge(i, ed5, sc2[:, 0], xs)
    return _tc_final(msg, den, hlin, convs[3][6].reshape(1, dos[3]), dos[3])


# speedup vs baseline: 35.3827x; 1.1392x over previous
import functools

import jax
import jax.numpy as jnp
from jax import lax
from jax.experimental import pallas as pl
from jax.experimental.pallas import tpu as pltpu
from jax.experimental.pallas import tpu_sc as plsc

N = 10000
E = 320000
NPAD = 10240
NTILE = 32
EPT = E // NTILE
CH = 80
R = 1024
ROWS_PER_TILE = NPAD // 16


def _f32(*shape):
    return jax.ShapeDtypeStruct(shape, jnp.float32)


_GDN = lax.GatherDimensionNumbers(
    offset_dims=(), collapsed_slice_dims=(0,), start_index_map=(0,))


def _bcast_lane(v16, lane):
    idx = jnp.full((16, 1), lane, jnp.int32)
    return lax.gather(v16, idx, _GDN, (1,),
                      mode=lax.GatherScatterMode.PROMISE_IN_BOUNDS)


def _pack_scores(sc):
    u = lax.bitcast_convert_type(sc, jnp.uint32)
    rne = u + jnp.uint32(0x7FFF) + ((u >> 16) & jnp.uint32(1))
    hi = rne & jnp.uint32(0xFFFF0000)
    lo = rne >> 16
    pk = hi | jnp.roll(lo, -1, axis=1)
    return lax.bitcast_convert_type(pk, jnp.int32)[:, :8]


def _tc_first_body(x_ref, wsrc_ref, wscp_ref, linw_ref, xs_ref, sc_ref, hlin_ref):
    h = x_ref[...]
    xs_ref[...] = jnp.dot(h, wsrc_ref[...], preferred_element_type=jnp.float32)
    sc_ref[...] = _pack_scores(
        jnp.dot(h, wscp_ref[...], preferred_element_type=jnp.float32))
    hlin_ref[...] = jnp.dot(h, linw_ref[...], preferred_element_type=jnp.float32)


def _finalize(msg_ref, den_ref, hlinp_ref, b_ref, dp):
    m = (msg_ref[0] + msg_ref[1])[:, :dp]
    dn = jnp.sum(den_ref[...], axis=0)
    return m / (dn[:, None] + 1e-30) + b_ref[...] + hlinp_ref[...]


def _tc_mid_body(dp, msg_ref, den_ref, hlinp_ref, b_ref, wsrc_ref, wscp_ref,
                 linw_ref, xs_ref, sc_ref, hlin_ref):
    h = jnp.maximum(_finalize(msg_ref, den_ref, hlinp_ref, b_ref, dp), 0.0)
    xs_ref[...] = jnp.dot(h, wsrc_ref[...], preferred_element_type=jnp.float32)
    sc_ref[...] = _pack_scores(
        jnp.dot(h, wscp_ref[...], preferred_element_type=jnp.float32))
    hlin_ref[...] = jnp.dot(h, linw_ref[...], preferred_element_type=jnp.float32)


def _tc_final_body(dp, msg_ref, den_ref, hlinp_ref, b_ref, out_ref):
    out_ref[...] = jax.nn.sigmoid(
        _finalize(msg_ref, den_ref, hlinp_ref, b_ref, dp))


def _tc_first(x, wsrc, wscp, linw, do):
    di = x.shape[1]
    return pl.pallas_call(
        _tc_first_body,
        grid=(pl.cdiv(N, R),),
        in_specs=[
            pl.BlockSpec((R, di), lambda i: (i, 0)),
            pl.BlockSpec((di, do), lambda i: (0, 0)),
            pl.BlockSpec((di, 128), lambda i: (0, 0)),
            pl.BlockSpec((di, do), lambda i: (0, 0)),
        ],
        out_specs=[
            pl.BlockSpec((R, do), lambda i: (i, 0)),
            pl.BlockSpec((R, 8), lambda i: (i, 0)),
            pl.BlockSpec((R, do), lambda i: (i, 0)),
        ],
        out_shape=[_f32(N, do),
                   jax.ShapeDtypeStruct((N, 8), jnp.int32),
                   _f32(N, do)],
    )(x, wsrc, wscp, linw)


def _tc_mid(msg, den, hlinp, b, wsrc, wscp, linw, dp, do):
    di = dp
    return pl.pallas_call(
        functools.partial(_tc_mid_body, dp),
        grid=(pl.cdiv(N, R),),
        in_specs=[
            pl.BlockSpec((2, R, 128), lambda i: (0, i, 0)),
            pl.BlockSpec((NTILE, R), lambda i: (0, i)),
            pl.BlockSpec((R, dp), lambda i: (i, 0)),
            pl.BlockSpec((1, dp), lambda i: (0, 0)),
            pl.BlockSpec((di, 128), lambda i: (0, 0)),
            pl.BlockSpec((di, 128), lambda i: (0, 0)),
            pl.BlockSpec((di, do), lambda i: (0, 0)),
        ],
        out_specs=[
            pl.BlockSpec((R, 128), lambda i: (i, 0)),
            pl.BlockSpec((R, 8), lambda i: (i, 0)),
            pl.BlockSpec((R, do), lambda i: (i, 0)),
        ],
        out_shape=[_f32(N, 128),
                   jax.ShapeDtypeStruct((N, 8), jnp.int32),
                   _f32(N, do)],
    )(msg, den, hlinp, b, wsrc, wscp, linw)


def _tc_final(msg, den, hlinp, b, dp):
    return pl.pallas_call(
        functools.partial(_tc_final_body, dp),
        grid=(pl.cdiv(N, R),),
        in_specs=[
            pl.BlockSpec((2, R, 128), lambda i: (0, i, 0)),
            pl.BlockSpec((NTILE, R), lambda i: (0, i)),
            pl.BlockSpec((R, dp), lambda i: (i, 0)),
            pl.BlockSpec((1, dp), lambda i: (0, 0)),
        ],
        out_specs=pl.BlockSpec((R, dp), lambda i: (i, 0)),
        out_shape=_f32(N, dp),
    )(msg, den, hlinp, b)


def _escore_body(ea_ref, bd_ref, out_ref):
    out_ref[...] = jnp.dot(ea_ref[...], bd_ref[...],
                           preferred_element_type=jnp.float32)


def _escore(ea2, bd):
    rows = ea2.shape[0]
    return pl.pallas_call(
        _escore_body,
        grid=(pl.cdiv(rows, R),),
        in_specs=[
            pl.BlockSpec((R, 128), lambda i: (i, 0)),
            pl.BlockSpec((128, 32), lambda i: (0, 0)),
        ],
        out_specs=pl.BlockSpec((R, 32), lambda i: (i, 0)),
        out_shape=_f32(rows, 32),
    )(ea2, bd)


T = EPT // CH
NBUF = 2


def _sc_body(li, ed_h, spk_h, xs_h, msg_o, den_o,
             spk_v, den_v, ec0, ec1, ex0, ex1, dc0, dc1, rows0, rows1,
             acc_s, es0, es1, gs0, gs1, ss0, ss1):
    cid = lax.axis_index("c")
    sid = lax.axis_index("s")
    w = sid * 2 + cid
    ecs = (ec0, ec1)
    exs = (ex0, ex1)
    dcs = (dc0, dc1)
    rows = (rows0, rows1)
    esems = (es0, es1)
    gsems = (gs0, gs1)
    ssems = (ss0, ss1)

    pltpu.sync_copy(spk_h, spk_v)

    zero16 = jnp.zeros((16,), jnp.float32)

    def _zden(k, carry):
        den_v[pl.ds(k * 16, 16)] = zero16
        return carry
    lax.fori_loop(0, NPAD // 16, _zden, 0)

    def _zrow(k, carry):
        rr = k // 8
        cc = (k - rr * 8) * 16
        rows0[rr, pl.ds(cc, 16)] = zero16
        return carry
    lax.fori_loop(0, CH * 8, _zrow, 0)
    for k in range(ROWS_PER_TILE // CH):
        pltpu.async_copy(rows0,
                         acc_s.at[pl.ds(sid * ROWS_PER_TILE + k * CH, CH)],
                         gs0)
    for k in range(ROWS_PER_TILE // CH):
        pltpu.make_async_copy(
            rows0, acc_s.at[pl.ds(sid * ROWS_PER_TILE, CH)], gs0).wait()
    plsc.subcore_barrier()

    def _ecopy_issue(t, b):
        pltpu.async_copy(ed_h.at[w, t], ecs[b], esems[b])

    def _ecopy_wait(b):
        pltpu.make_async_copy(ed_h.at[0, 0], ecs[b], esems[b]).wait()

    def _gather_issue(b):
        pltpu.async_copy(xs_h.at[ecs[b].at[0]], rows[b], gsems[b])

    def _gather_wait(b):
        pltpu.make_async_copy(xs_h.at[pl.ds(0, CH)], rows[b],
                              gsems[b]).wait()

    def _scatter_issue(b):
        pltpu.async_copy(rows[b], acc_s.at[dcs[b]], ssems[b], add=True)

    def _scatter_wait(b):
        pltpu.make_async_copy(xs_h.at[pl.ds(0, CH)], rows[b],
                              ssems[b]).wait()

    hi_mask = jnp.full((16,), -65536, jnp.int32)

    def _scores(b):
        ec = ecs[b]
        exc = exs[b]
        dc = dcs[b]
        for g in range(CH // 16):
            s16 = ec[0, pl.ds(g * 16, 16)]
            d16 = ec[1, pl.ds(g * 16, 16)]
            el16 = plsc.bitcast(ec[2 + li, pl.ds(g * 16, 16)], jnp.float32)
            ws = plsc.load_gather(spk_v, [s16])
            wd = plsc.load_gather(spk_v, [d16])
            ssrc = plsc.bitcast(ws & hi_mask, jnp.float32)
            sdst = plsc.bitcast(wd << 16, jnp.float32)
            a = ssrc + sdst + el16
            a = jnp.where(a >= 0.0, a, 0.2 * a)
            ex = jnp.exp(a)
            exc[pl.ds(g * 16, 16)] = ex
            dc[pl.ds(g * 16, 16)] = d16
            plsc.addupdate_scatter(den_v, [d16], ex)

    def _scale(b):
        rv = rows[b]
        exc = exs[b]

        def _sg(g, c2):
            ex16 = exc[pl.ds(g * 16, 16)]
            for l in range(16):
                exj = _bcast_lane(ex16, l)
                j = g * 16 + l
                for c in range(8):
                    rv[j, pl.ds(c * 16, 16)] = (
                        rv[j, pl.ds(c * 16, 16)] * exj)
            return c2
        lax.fori_loop(0, CH // 16, _sg, 0)

    def _pipe(t, b):
        nb = (b + 1) % NBUF
        _gather_wait(b)

        @pl.when(t >= 1)
        def _():
            _scatter_wait(nb)

        @pl.when(t + 1 < T)
        def _():
            _ecopy_wait(nb)
            _gather_issue(nb)
        _scale(b)
        _scatter_issue(b)

        @pl.when(t + 1 < T)
        def _():
            _scores(nb)

        @pl.when(t + 2 < T)
        def _():
            _ecopy_issue(t + 2, b)

    _ecopy_issue(0, 0)
    _ecopy_wait(0)
    _gather_issue(0)
    _scores(0)
    _ecopy_issue(1, 1)
    _pipe(0, 0)

    def _pair(k, carry):
        t0 = 1 + k * 2
        for j in range(2):
            _pipe(t0 + j, (1 + j) % NBUF)
        return carry
    lax.fori_loop(0, (T - 1) // 2, _pair, 0)
    _scatter_wait((T - 1) % NBUF)

    plsc.subcore_barrier()
    pltpu.sync_copy(acc_s.at[pl.ds(sid * ROWS_PER_TILE, ROWS_PER_TILE)],
                    msg_o.at[cid, pl.ds(sid * ROWS_PER_TILE, ROWS_PER_TILE)])
    pltpu.sync_copy(den_v, den_o.at[w])


def _sc_edge(li, ed5, spk, xs):
    mesh = plsc.VectorSubcoreMesh(core_axis_name="c", subcore_axis_name="s")
    kern = pl.kernel(
        functools.partial(_sc_body, li),
        mesh=mesh,
        compiler_params=pltpu.CompilerParams(needs_layout_passes=False),
        out_type=[_f32(2, NPAD, 128), _f32(NTILE, NPAD)],
        scratch_types=[
            pltpu.VMEM((N,), jnp.int32),
            pltpu.VMEM((NPAD,), jnp.float32),
            pltpu.VMEM((6, CH), jnp.int32),
            pltpu.VMEM((6, CH), jnp.int32),
            pltpu.VMEM((CH,), jnp.float32),
            pltpu.VMEM((CH,), jnp.float32),
            pltpu.VMEM((CH,), jnp.int32),
            pltpu.VMEM((CH,), jnp.int32),
            pltpu.VMEM((CH, 128), jnp.float32),
            pltpu.VMEM((CH, 128), jnp.float32),
            pltpu.VMEM_SHARED((NPAD, 128), jnp.float32),
            pltpu.SemaphoreType.DMA,
            pltpu.SemaphoreType.DMA,
            pltpu.SemaphoreType.DMA,
            pltpu.SemaphoreType.DMA,
            pltpu.SemaphoreType.DMA,
            pltpu.SemaphoreType.DMA,
        ],
    )
    return kern(ed5, spk, xs)


def kernel(x, edge_index, edge_attr,
           conv1_Wsrc, conv1_Wdst, conv1_We, conv1_asrc, conv1_adst, conv1_ae, conv1_b, lin1_W,
           conv2_Wsrc, conv2_Wdst, conv2_We, conv2_asrc, conv2_adst, conv2_ae, conv2_b, lin2_W,
           conv3_Wsrc, conv3_Wdst, conv3_We, conv3_asrc, conv3_adst, conv3_ae, conv3_b, lin3_W,
           conv4_Wsrc, conv4_Wdst, conv4_We, conv4_asrc, conv4_adst, conv4_ae, conv4_b, lin4_W):
    convs = [
        (conv1_Wsrc, conv1_Wdst, conv1_We, conv1_asrc, conv1_adst, conv1_ae, conv1_b, lin1_W),
        (conv2_Wsrc, conv2_Wdst, conv2_We, conv2_asrc, conv2_adst, conv2_ae, conv2_b, lin2_W),
        (conv3_Wsrc, conv3_Wdst, conv3_We, conv3_asrc, conv3_adst, conv3_ae, conv3_b, lin3_W),
        (conv4_Wsrc, conv4_Wdst, conv4_We, conv4_asrc, conv4_adst, conv4_ae, conv4_b, lin4_W),
    ]
    srcp = edge_index[0].reshape(NTILE, T, 1, CH)
    dstp = edge_index[1].reshape(NTILE, T, 1, CH)

    wscps = []
    for (Wsrc, Wdst, We, asrc, adst, ae, b, linW) in convs:
        wsa = Wsrc @ asrc
        wdd = Wdst @ adst
        wscp = jnp.zeros((Wsrc.shape[0], 128), jnp.float32)
        wscp = wscp.at[:, 0].set(wsa).at[:, 1].set(wdd)
        wscps.append(wscp)
    blk = jnp.stack([c[2] @ c[5] for c in convs], axis=1)
    bd = jnp.kron(jnp.eye(8, dtype=jnp.float32), blk)

    es4 = _escore(edge_attr.reshape(E // 8, 128), bd).reshape(E, 4)

    dos = [128, 128, 128, 64]
    xs, sc2, hlin = _tc_first(x, convs[0][0], wscps[0], convs[0][7], 128)
    el4t = lax.bitcast_convert_type(es4, jnp.int32).reshape(
        NTILE, T, CH, 4).transpose(0, 1, 3, 2)
    ed5 = jnp.concatenate([srcp, dstp, el4t], axis=2)
    msg, den = _sc_edge(0, ed5, sc2[:, 0], xs)
    for i in (1, 2, 3):
        dp, do = dos[i - 1], dos[i]
        b_prev = convs[i - 1][6].reshape(1, dp)
        wsrc = convs[i][0]
        if wsrc.shape[1] < 128:
            wsrc = jnp.pad(wsrc, ((0, 0), (0, 128 - wsrc.shape[1])))
        xs, sc2, hlin = _tc_mid(msg, den, hlin, b_prev,
                                wsrc, wscps[i], convs[i][7], dp, do)
        msg, den = _sc_edge(i, ed5, sc2[:, 0], xs)
    return _tc_final(msg, den, hlin, convs[3][6].reshape(1, dos[3]), dos[3])
